# baseline (speedup 1.0000x reference)
import functools

import jax
import jax.numpy as jnp
from jax import lax
from jax.experimental import pallas as pl
from jax.experimental.pallas import tpu as pltpu
from jax.experimental.pallas import tpu_sc as plsc

_BANK = 100000
_DIM = 128
_BATCH = 16384

_BLK = 8192
_NB = (_BANK + _BLK - 1) // _BLK
_SPLIT = _BATCH // _BLK


def _fb_body(feat_ref, bank_ref, out_fb_ref):
    i = pl.program_id(0)

    @pl.when(i < _SPLIT)
    def _():
        out_fb_ref[...] = feat_ref[...]

    @pl.when(i >= _SPLIT)
    def _():
        out_fb_ref[...] = bank_ref[...]


def _fb_copy(features, feature_bank):
    return pl.pallas_call(
        _fb_body,
        grid=(_NB,),
        in_specs=[
            pl.BlockSpec((_BLK, _DIM), lambda i: (jnp.minimum(i, _SPLIT - 1), 0)),
            pl.BlockSpec((_BLK, _DIM), lambda i: (jnp.maximum(i, _SPLIT), 0)),
        ],
        out_specs=pl.BlockSpec((_BLK, _DIM), lambda i: (i, 0)),
        out_shape=jax.ShapeDtypeStruct((_BANK, _DIM), feature_bank.dtype),
    )(features, feature_bank)


_LCHUNK = 4000
_NCHUNK = _BANK // _LCHUNK
_LSPLIT = _BATCH // _LCHUNK
_LOFF = _BATCH - _LSPLIT * _LCHUNK


def _lb_body(lab_hbm, lbank_hbm, out_hbm, buf):
    wid = lax.axis_index("s") * 2 + lax.axis_index("c")
    base = wid * _LCHUNK

    @pl.when(wid < _LSPLIT)
    def _():
        pltpu.sync_copy(lab_hbm.at[pl.ds(base, _LCHUNK)], buf)
        pltpu.sync_copy(buf, out_hbm.at[pl.ds(base, _LCHUNK)])

    @pl.when(wid == _LSPLIT)
    def _():
        pltpu.sync_copy(lab_hbm.at[pl.ds(base, _LOFF)], buf.at[pl.ds(0, _LOFF)])
        pltpu.sync_copy(lbank_hbm.at[pl.ds(base + _LOFF, _LCHUNK - _LOFF)],
                        buf.at[pl.ds(_LOFF, _LCHUNK - _LOFF)])
        pltpu.sync_copy(buf, out_hbm.at[pl.ds(base, _LCHUNK)])

    @pl.when(jnp.logical_and(wid > _LSPLIT, wid < _NCHUNK))
    def _():
        pltpu.sync_copy(lbank_hbm.at[pl.ds(base, _LCHUNK)], buf)
        pltpu.sync_copy(buf, out_hbm.at[pl.ds(base, _LCHUNK)])


def _lb_copy(labels, label_bank):
    run = functools.partial(
        pl.kernel,
        out_type=jax.ShapeDtypeStruct((_BANK,), label_bank.dtype),
        mesh=plsc.VectorSubcoreMesh(core_axis_name="c", subcore_axis_name="s"),
        scratch_types=[pltpu.VMEM((_LCHUNK,), label_bank.dtype)],
    )(_lb_body)
    return run(labels, label_bank)


def kernel(features, labels, feature_bank, label_bank):
    out_fb = _fb_copy(features, feature_bank)
    out_lb = _lb_copy(labels, label_bank)
    return out_fb, out_lb


# speedup vs baseline: 1.0029x; 1.0029x over previous
import functools

import jax
import jax.numpy as jnp
from jax import lax
from jax.experimental import pallas as pl
from jax.experimental.pallas import tpu as pltpu
from jax.experimental.pallas import tpu_sc as plsc

_BANK = 100000
_DIM = 128
_BATCH = 16384

_BLK = 8192
_NB = (_BANK + _BLK - 1) // _BLK
_SPLIT = _BATCH // _BLK


def _fb_body(feat_ref, bank_ref, out_fb_ref):
    i = pl.program_id(0)

    @pl.when(i < _SPLIT)
    def _():
        out_fb_ref[...] = feat_ref[...]

    @pl.when(i >= _SPLIT)
    def _():
        out_fb_ref[...] = bank_ref[...]


def _fb_copy(features, feature_bank):
    return pl.pallas_call(
        _fb_body,
        grid=(_NB,),
        in_specs=[
            pl.BlockSpec((_BLK, _DIM), lambda i: (jnp.minimum(i, _SPLIT - 1), 0)),
            pl.BlockSpec((_BLK, _DIM), lambda i: (jnp.maximum(i, _SPLIT), 0)),
        ],
        out_specs=pl.BlockSpec((_BLK, _DIM), lambda i: (i, 0)),
        out_shape=jax.ShapeDtypeStruct((_BANK, _DIM), feature_bank.dtype),
    )(features, feature_bank)


_LCHUNK = 4000
_NCHUNK = _BANK // _LCHUNK
_LSPLIT = _BATCH // _LCHUNK
_LOFF = _BATCH - _LSPLIT * _LCHUNK


def _lb_body(lab_hbm, lbank_hbm, out_hbm, buf):
    wid = lax.axis_index("s") * 2 + lax.axis_index("c")
    base = wid * _LCHUNK

    @pl.when(wid < _LSPLIT)
    def _():
        pltpu.sync_copy(lab_hbm.at[pl.ds(base, _LCHUNK)], buf)
        pltpu.sync_copy(buf, out_hbm.at[pl.ds(base, _LCHUNK)])

    @pl.when(wid == _LSPLIT)
    def _():
        pltpu.sync_copy(lab_hbm.at[pl.ds(base, _LOFF)], buf.at[pl.ds(0, _LOFF)])
        pltpu.sync_copy(lbank_hbm.at[pl.ds(base + _LOFF, _LCHUNK - _LOFF)],
                        buf.at[pl.ds(_LOFF, _LCHUNK - _LOFF)])
        pltpu.sync_copy(buf, out_hbm.at[pl.ds(base, _LCHUNK)])

    @pl.when(jnp.logical_and(wid > _LSPLIT, wid < _NCHUNK))
    def _():
        pltpu.sync_copy(lbank_hbm.at[pl.ds(base, _LCHUNK)], buf)
        pltpu.sync_copy(buf, out_hbm.at[pl.ds(base, _LCHUNK)])


def _lb_copy(labels, label_bank):
    run = functools.partial(
        pl.kernel,
        out_type=jax.ShapeDtypeStruct((_BANK,), label_bank.dtype),
        mesh=plsc.VectorSubcoreMesh(core_axis_name="c", subcore_axis_name="s"),
        scratch_types=[pltpu.VMEM((_LCHUNK,), label_bank.dtype)],
    )(_lb_body)
    return run(labels, label_bank)


def kernel(features, labels, feature_bank, label_bank):
    out_lb = _lb_copy(labels, label_bank)
    out_fb = _fb_copy(features, feature_bank)
    return out_fb, out_lb
